# baseline (speedup 1.0000x reference)
import functools

import jax
import jax.numpy as jnp
from jax import lax
from jax.experimental import pallas as pl
from jax.experimental.pallas import tpu as pltpu
from jax.experimental.pallas import tpu_sc as plsc

B = 16384
D = 64
NC = 2
NS = 16
L = 16
NW = NC * NS
BPW = B // NW
CHUNK = 128
NCHUNK = BPW // CHUNK
NGROUP = BPW // L


def _body(user_h, item_h, eu_h, ev_h, bu_h, bv_h, out_h,
          idx_u, idx_v, rows_u, rows_v, b_u, b_v, out_v, sem):
    wid = lax.axis_index("s") * NC + lax.axis_index("c")

    pltpu.sync_copy(user_h.at[pl.ds(wid * NCHUNK, NCHUNK)], idx_u)
    pltpu.sync_copy(item_h.at[pl.ds(wid * NCHUNK, NCHUNK)], idx_v)

    copies = []
    for j in range(NCHUNK):
        dst = pl.ds(j * CHUNK, CHUNK)
        copies.append(pltpu.async_copy(eu_h.at[idx_u.at[j]], rows_u.at[dst], sem))
        copies.append(pltpu.async_copy(ev_h.at[idx_v.at[j]], rows_v.at[dst], sem))
        copies.append(pltpu.async_copy(bu_h.at[idx_u.at[j]], b_u.at[dst], sem))
        copies.append(pltpu.async_copy(bv_h.at[idx_v.at[j]], b_v.at[dst], sem))
    for c in copies:
        c.wait()

    lanes = lax.iota(jnp.int32, L)

    def group(g, _):
        rows = g * L + lanes
        acc = b_u[pl.ds(g * L, L)] + b_v[pl.ds(g * L, L)]
        for d in range(D):
            col = jnp.full((L,), d, jnp.int32)
            acc = acc + plsc.load_gather(rows_u, [rows, col]) * \
                plsc.load_gather(rows_v, [rows, col])
        out_v[pl.ds(g * L, L)] = jnp.clip(acc, 0.0, 5.0)
        return 0

    lax.fori_loop(0, NGROUP, group, 0)

    pltpu.sync_copy(out_v, out_h.at[pl.ds(wid * BPW, BPW)])


@functools.partial(
    pl.kernel,
    out_type=jax.ShapeDtypeStruct((B,), jnp.float32),
    mesh=plsc.VectorSubcoreMesh(core_axis_name="c", subcore_axis_name="s"),
    scratch_types=[
        pltpu.VMEM((NCHUNK, CHUNK), jnp.int32),
        pltpu.VMEM((NCHUNK, CHUNK), jnp.int32),
        pltpu.VMEM((BPW, D), jnp.float32),
        pltpu.VMEM((BPW, D), jnp.float32),
        pltpu.VMEM((BPW,), jnp.float32),
        pltpu.VMEM((BPW,), jnp.float32),
        pltpu.VMEM((BPW,), jnp.float32),
        pltpu.SemaphoreType.DMA,
    ],
    compiler_params=pltpu.CompilerParams(
        needs_layout_passes=False, use_tc_tiling_on_sc=False),
)
def _sc_svd(user_h, item_h, eu_h, ev_h, bu_h, bv_h, out_h,
            idx_u, idx_v, rows_u, rows_v, b_u, b_v, out_v, sem):
    _body(user_h, item_h, eu_h, ev_h, bu_h, bv_h, out_h,
          idx_u, idx_v, rows_u, rows_v, b_u, b_v, out_v, sem)


def kernel(user, item, embed_user, embed_item, bias_user, bias_item):
    user2 = user.reshape(NW * NCHUNK, CHUNK).astype(jnp.int32)
    item2 = item.reshape(NW * NCHUNK, CHUNK).astype(jnp.int32)
    bu = bias_user.reshape(-1)
    bv = bias_item.reshape(-1)
    return _sc_svd(user2, item2, embed_user, embed_item, bu, bv)
